# Initial kernel scaffold; baseline (speedup 1.0000x reference)
import jax
import jax.numpy as jnp
from jax.experimental import pallas as pl


def kernel(x, edge_index, batch, params):
    raise NotImplementedError("write your pallas kernel here")


# speedup vs baseline: 2.1579x; 2.1579x over previous
import functools

import jax
import jax.numpy as jnp
from jax import lax
from jax.experimental import pallas as pl
from jax.experimental.pallas import tpu as pltpu
from jax.experimental.pallas import tpu_sc as plsc

N = 10000
E = 160000
F_IN = 5
H = 512
L = 4
C = 10
G = 64

NC = 4
CW = H // NC
BR = 2000
NBLK = N // BR

NSC = 2
NSUB = 16
EPW = E // NSUB
EB = 400
NEB = EPW // EB
NPAD = 10240
HALF = NPAD // 2
JROWS = NSUB * 16
ACCR = HALF + JROWS
ZR = ACCR // NSUB
WR = HALF // NSUB

f32 = jnp.float32


def _agg_body(h_hbm, src_hbm, dst_hbm, zeros_hbm, out_hbm,
              src_v, dst_v, rows_v, acc, sem):
    c = lax.axis_index("c")
    s = lax.axis_index("s")
    nbase = c * HALF
    junk = HALF + s * 16 + lax.iota(jnp.int32, 16)
    for chunk in range(NC):
        pltpu.sync_copy(zeros_hbm, acc.at[pl.ds(s * ZR, ZR)])
        plsc.subcore_barrier()
        ebase = s * EPW

        @pl.loop(0, NEB)
        def _edge_block(b):
            off = ebase + b * EB
            pltpu.sync_copy(src_hbm.at[pl.ds(chunk * E + off, EB)], src_v)
            pltpu.sync_copy(dst_hbm.at[pl.ds(off, EB)], dst_v)
            for j in range(EB // 16):
                d = dst_v[pl.ds(j * 16, 16)] - nbase
                ok = (d >= 0) & (d < HALF)
                dst_v[pl.ds(j * 16, 16)] = jnp.where(ok, d, junk)
            pltpu.async_copy(h_hbm.at[src_v], rows_v, sem).wait()
            pltpu.sync_copy(rows_v, acc.at[dst_v], add=True)

        plsc.subcore_barrier()
        pltpu.sync_copy(acc.at[pl.ds(s * WR, WR)],
                        out_hbm.at[pl.ds(chunk * NPAD + nbase + s * WR, WR)])
        plsc.subcore_barrier()


_agg_call = pl.kernel(
    _agg_body,
    out_type=jax.ShapeDtypeStruct((NC * NPAD, CW), f32),
    mesh=plsc.VectorSubcoreMesh(core_axis_name="c", subcore_axis_name="s"),
    scratch_types=[
        pltpu.VMEM((EB,), jnp.int32),
        pltpu.VMEM((EB,), jnp.int32),
        pltpu.VMEM((EB, CW), f32),
        pltpu.VMEM_SHARED((ACCR, CW), f32),
        pltpu.SemaphoreType.DMA,
    ],
    name="sc_segment_sum",
)


def _proj_body(x_ref, w_ref, p_ref):
    p = jnp.dot(x_ref[...], w_ref[...], preferred_element_type=f32)
    for c in range(NC):
        p_ref[c] = p[:, c * CW:(c + 1) * CW]


def _stats_accum(i, y, acc_s, acc_ss, st_ref):
    @pl.when(i == 0)
    def _():
        acc_s[...] = jnp.zeros_like(acc_s)
        acc_ss[...] = jnp.zeros_like(acc_ss)

    acc_s[...] += jnp.sum(y, axis=0, keepdims=True)
    acc_ss[...] += jnp.sum(y * y, axis=0, keepdims=True)

    @pl.when(i == NBLK - 1)
    def _():
        st_ref[0:1, :] = acc_s[...]
        st_ref[1:2, :] = acc_ss[...]


def _addagg_body(p_ref, agg_ref, b1_ref, eps_ref, y_ref, st_ref, acc_s, acc_ss):
    i = pl.program_id(0)
    ep = 1.0 + eps_ref[0, 0]
    parts = [ep * p_ref[c] + agg_ref[c] for c in range(NC)]
    y = jnp.concatenate(parts, axis=1) + b1_ref[...]
    y_ref[...] = y
    _stats_accum(i, y, acc_s, acc_ss, st_ref)


def _conv_mm1_body(h_ref, agg_ref, w1_ref, b1_ref, eps_ref,
                   y_ref, st_ref, acc_s, acc_ss):
    i = pl.program_id(0)
    ep = 1.0 + eps_ref[0, 0]
    y = jnp.zeros((BR, H), f32)
    for c in range(NC):
        z = ep * h_ref[c] + agg_ref[c]
        y += jnp.dot(z, w1_ref[pl.ds(c * CW, CW), :], preferred_element_type=f32)
    y = y + b1_ref[...]
    y_ref[...] = y
    _stats_accum(i, y, acc_s, acc_ss, st_ref)


def _norm_coefs(st_ref, g_ref, be_ref):
    m = st_ref[0:1, :] * (1.0 / N)
    ey2 = st_ref[1:2, :] * (1.0 / N)
    var = ey2 - m * m
    rstd = lax.rsqrt(var + 1e-5)
    scale = rstd * g_ref[...]
    shift = be_ref[...] - m * scale
    return scale, shift


def _conv_mm2_body(y1_ref, st_ref, g_ref, be_ref, w2_ref, b2_ref,
                   y2_ref, st2_ref, acc_s, acc_ss):
    i = pl.program_id(0)
    scale, shift = _norm_coefs(st_ref, g_ref, be_ref)
    z = jnp.maximum(y1_ref[...] * scale + shift, 0.0)
    y = jnp.dot(z, w2_ref[...], preferred_element_type=f32) + b2_ref[...]
    y2_ref[...] = y
    _stats_accum(i, y, acc_s, acc_ss, st2_ref)


def _norm_relu_body(y2_ref, st_ref, g_ref, be_ref, h_ref):
    scale, shift = _norm_coefs(st_ref, g_ref, be_ref)
    z = jnp.maximum(y2_ref[...] * scale + shift, 0.0)
    for c in range(NC):
        h_ref[c] = z[:, c * CW:(c + 1) * CW]


def _final_body(h1_ref, h2_ref, h3_ref, h4_ref, batch_ref,
                w_ref, b_ref, pw_ref, pb_ref, out_ref, pool_ref):
    i = pl.program_id(0)
    hrefs = [h1_ref, h2_ref, h3_ref, h4_ref]
    hl = jnp.zeros((BR, H), f32)
    for l in range(L):
        for c in range(NC):
            k = l * NC + c
            hl += jnp.dot(hrefs[l][c], w_ref[pl.ds(k * CW, CW), :],
                          preferred_element_type=f32)
    hl = jnp.maximum(hl + b_ref[...], 0.0)
    bvals = batch_ref[0, 0, :].reshape(BR, 1)
    gids = lax.broadcasted_iota(jnp.int32, (1, G), 1)
    onehot = (bvals == gids).astype(f32)
    contrib = lax.dot_general(onehot, hl, (((0,), (0,)), ((), ())),
                              preferred_element_type=f32)

    @pl.when(i == 0)
    def _():
        pool_ref[...] = jnp.zeros_like(pool_ref)

    pool_ref[...] += contrib

    @pl.when(i == NBLK - 1)
    def _():
        out_ref[...] = (jnp.dot(pool_ref[...], pw_ref[...],
                                preferred_element_type=f32) + pb_ref[...])


def _chunk_spec():
    return pl.BlockSpec((NC, BR, CW), lambda i: (0, i, 0))


def _agg_spec():
    return pl.BlockSpec((NC, BR, CW), lambda i: (0, i, 0))


def _row_spec():
    return pl.BlockSpec((BR, H), lambda i: (i, 0))


def _full(shape):
    return pl.BlockSpec(shape, lambda i: tuple(0 for _ in shape))


def _stats_scratch():
    return [pltpu.VMEM((1, H), f32), pltpu.VMEM((1, H), f32)]


_proj = pl.pallas_call(
    _proj_body,
    grid=(NBLK,),
    in_specs=[pl.BlockSpec((BR, F_IN), lambda i: (i, 0)), _full((F_IN, H))],
    out_specs=_chunk_spec(),
    out_shape=jax.ShapeDtypeStruct((NC, N, CW), f32),
)

_addagg = pl.pallas_call(
    _addagg_body,
    grid=(NBLK,),
    in_specs=[_chunk_spec(), _agg_spec(), _full((1, H)), _full((1, 1))],
    out_specs=[_row_spec(), _full((2, H))],
    out_shape=[jax.ShapeDtypeStruct((N, H), f32),
               jax.ShapeDtypeStruct((2, H), f32)],
    scratch_shapes=_stats_scratch(),
)

_conv_mm1 = pl.pallas_call(
    _conv_mm1_body,
    grid=(NBLK,),
    in_specs=[_chunk_spec(), _agg_spec(), _full((H, H)), _full((1, H)),
              _full((1, 1))],
    out_specs=[_row_spec(), _full((2, H))],
    out_shape=[jax.ShapeDtypeStruct((N, H), f32),
               jax.ShapeDtypeStruct((2, H), f32)],
    scratch_shapes=_stats_scratch(),
)

_conv_mm2 = pl.pallas_call(
    _conv_mm2_body,
    grid=(NBLK,),
    in_specs=[_row_spec(), _full((2, H)), _full((1, H)), _full((1, H)),
              _full((H, H)), _full((1, H))],
    out_specs=[_row_spec(), _full((2, H))],
    out_shape=[jax.ShapeDtypeStruct((N, H), f32),
               jax.ShapeDtypeStruct((2, H), f32)],
    scratch_shapes=_stats_scratch(),
)

_norm_relu = pl.pallas_call(
    _norm_relu_body,
    grid=(NBLK,),
    in_specs=[_row_spec(), _full((2, H)), _full((1, H)), _full((1, H))],
    out_specs=_chunk_spec(),
    out_shape=jax.ShapeDtypeStruct((NC, N, CW), f32),
)

_final = pl.pallas_call(
    _final_body,
    grid=(NBLK,),
    in_specs=[_chunk_spec(), _chunk_spec(), _chunk_spec(), _chunk_spec(),
              pl.BlockSpec((1, 1, BR), lambda i: (i, 0, 0)),
              _full((L * H, H)), _full((1, H)), _full((H, C)), _full((1, C))],
    out_specs=_full((G, C)),
    out_shape=jax.ShapeDtypeStruct((G, C), f32),
    scratch_shapes=[pltpu.VMEM((G, H), f32)],
)


def _row(v):
    return v.reshape(1, -1)


def kernel(x, edge_index, batch, params):
    src = edge_index[0]
    dst = edge_index[1]
    src4 = (src[None, :] + (jnp.arange(NC, dtype=jnp.int32) * N)[:, None])
    src4 = src4.reshape(NC * E)
    zeros = jnp.zeros((ZR, CW), f32)
    batch_r = batch.reshape(NBLK, 1, BR)

    def run_agg(h_chunks):
        flat = h_chunks.reshape(NC * N, CW)
        return _agg_call(flat, src4, dst, zeros).reshape(NC, NPAD, CW)

    hs = []
    h_chunks = None
    for li in range(L):
        p = params['convs'][li]
        eps = p['eps'].reshape(1, 1)
        if li == 0:
            pch = _proj(x, p['W1'])
            agg = run_agg(pch)
            y1, st1 = _addagg(pch, agg, _row(p['b1']), eps)
        else:
            agg = run_agg(h_chunks)
            y1, st1 = _conv_mm1(h_chunks, agg, p['W1'], _row(p['b1']), eps)
        y2, st2 = _conv_mm2(y1, st1, _row(p['g1']), _row(p['be1']),
                            p['W2'], _row(p['b2']))
        h_chunks = _norm_relu(y2, st2, _row(p['g2']), _row(p['be2']))
        hs.append(h_chunks)

    out = _final(hs[0], hs[1], hs[2], hs[3], batch_r,
                 params['lin1_W'], _row(params['lin1_b']),
                 params['pred_W'], _row(params['pred_b']))
    return out


# speedup vs baseline: 3.2279x; 1.4958x over previous
import functools

import jax
import jax.numpy as jnp
from jax import lax
from jax.experimental import pallas as pl
from jax.experimental.pallas import tpu as pltpu
from jax.experimental.pallas import tpu_sc as plsc

N = 10000
E = 160000
F_IN = 5
H = 512
L = 4
C = 10
G = 64

NC = 4
CW = H // NC
BR = 2000
NBLK = N // BR

NSC = 2
NSUB = 16
EPW = E // NSUB
EB = 400
NEB = EPW // EB
NPAD = 10240
HALF = NPAD // 2
JROWS = NSUB * 16
ACCR = HALF + JROWS
ZR = ACCR // NSUB
WR = HALF // NSUB
EC = E + NSUB * EB
BMAX = EC // EB
PCAP = EPW + EB

f32 = jnp.float32


def _part_body(src_hbm, dst_hbm, psrc_hbm, pdst_hbm, nb_hbm,
               src_v, dst_v, bs, bd, stage_v, tbl_v, table):
    c = lax.axis_index("c")
    s = lax.axis_index("s")
    lane = lax.iota(jnp.int32, 16)
    keep_low = c == 0
    ebase = s * EPW

    @pl.loop(0, PCAP // 16)
    def _fill(i):
        bs[pl.ds(i * 16, 16)] = lane
        bd[pl.ds(i * 16, 16)] = jnp.full((16,), 1 << 20, jnp.int32)

    @pl.loop(0, NEB, init_carry=jnp.zeros((16,), jnp.int32))
    def _count(b, cnt):
        pltpu.sync_copy(dst_hbm.at[pl.ds(ebase + b * EB, EB)], dst_v)
        for j in range(EB // 16):
            low = dst_v[pl.ds(j * 16, 16)] < HALF
            m = jnp.where(keep_low, low, ~low)
            cnt = cnt + plsc.all_reduce_population_count(m)
        return cnt

    stage_v[...] = _count
    pltpu.sync_copy(stage_v, table.at[pl.ds(s * 16, 16)])
    plsc.subcore_barrier()
    pltpu.sync_copy(table, tbl_v)

    nblk_before = jnp.int32(0)
    nblk_total = jnp.int32(0)
    for j in range(NSUB):
        bj = (tbl_v[pl.ds(j * 16, 16)][0] + (EB - 1)) // EB
        nblk_total = nblk_total + bj
        nblk_before = nblk_before + jnp.where(s > j, bj, 0)
    my_off = c * EC + nblk_before * EB

    @pl.loop(0, NEB, init_carry=jnp.int32(0))
    def _compact(b, off):
        pltpu.sync_copy(src_hbm.at[pl.ds(ebase + b * EB, EB)], src_v)
        pltpu.sync_copy(dst_hbm.at[pl.ds(ebase + b * EB, EB)], dst_v)
        for j in range(EB // 16):
            d16 = dst_v[pl.ds(j * 16, 16)]
            s16 = src_v[pl.ds(j * 16, 16)]
            low = d16 < HALF
            m = jnp.where(keep_low, low, ~low)
            plsc.store_compressed(bs.at[pl.ds(off, 16)], s16, mask=m)
            plsc.store_compressed(bd.at[pl.ds(off, 16)], d16, mask=m)
            stage_v[...] = plsc.all_reduce_population_count(m)
            off = off + stage_v[...][0]
        return off

    bs[pl.ds(_compact, 16)] = lane
    bd[pl.ds(_compact, 16)] = jnp.full((16,), 1 << 20, jnp.int32)

    nblk = (_compact + (EB - 1)) // EB

    @pl.loop(0, nblk)
    def _flush(b):
        pltpu.sync_copy(bs.at[pl.ds(b * EB, EB)],
                        psrc_hbm.at[pl.ds(my_off + b * EB, EB)])
        pltpu.sync_copy(bd.at[pl.ds(b * EB, EB)],
                        pdst_hbm.at[pl.ds(my_off + b * EB, EB)])

    @pl.when(s == 0)
    def _():
        stage_v[...] = jnp.where(lane == 0, nblk_total, 0)
        pltpu.sync_copy(stage_v, nb_hbm.at[pl.ds(c * 16, 16)])


_part_call = pl.kernel(
    _part_body,
    out_type=(jax.ShapeDtypeStruct((NSC * EC,), jnp.int32),
              jax.ShapeDtypeStruct((NSC * EC,), jnp.int32),
              jax.ShapeDtypeStruct((NSC * 16,), jnp.int32)),
    mesh=plsc.VectorSubcoreMesh(core_axis_name="c", subcore_axis_name="s"),
    compiler_params=pltpu.CompilerParams(needs_layout_passes=False),
    scratch_types=[
        pltpu.VMEM((EB,), jnp.int32),
        pltpu.VMEM((EB,), jnp.int32),
        pltpu.VMEM((PCAP,), jnp.int32),
        pltpu.VMEM((PCAP,), jnp.int32),
        pltpu.VMEM((16,), jnp.int32),
        pltpu.VMEM((NSUB * 16,), jnp.int32),
        pltpu.VMEM_SHARED((NSUB * 16,), jnp.int32),
    ],
    name="sc_edge_partition",
)


def _agg_body(h_hbm, src_hbm, dst_hbm, nb_hbm, zeros_hbm, out_hbm,
              src_v, dst_v, rows_v, nb_v, acc, sem):
    c = lax.axis_index("c")
    s = lax.axis_index("s")
    lane = lax.iota(jnp.int32, 16)
    nbase = c * HALF
    junk = HALF + s * 16 + lane
    pltpu.sync_copy(nb_hbm.at[pl.ds(c * 16, 16)], nb_v)
    bg = nb_v[...][0]
    bt = (bg + (NSUB - 1)) // NSUB
    lo = jnp.minimum(s * bt, bg)
    hi = jnp.minimum(lo + bt, bg)
    gbase = c * EC

    for chunk in range(NC):
        pltpu.sync_copy(zeros_hbm, acc.at[pl.ds(s * ZR, ZR)])
        plsc.subcore_barrier()

        @pl.loop(lo, hi)
        def _edge_block(b):
            off = gbase + b * EB
            pltpu.sync_copy(src_hbm.at[pl.ds(off, EB)], src_v)
            pltpu.sync_copy(dst_hbm.at[pl.ds(off, EB)], dst_v)
            for j in range(EB // 16):
                d = dst_v[pl.ds(j * 16, 16)] - nbase
                ok = (d >= 0) & (d < HALF)
                dst_v[pl.ds(j * 16, 16)] = jnp.where(ok, d, junk)
                src_v[pl.ds(j * 16, 16)] = (src_v[pl.ds(j * 16, 16)]
                                            + chunk * N)
            pltpu.async_copy(h_hbm.at[src_v], rows_v, sem).wait()
            pltpu.sync_copy(rows_v, acc.at[dst_v], add=True)

        plsc.subcore_barrier()
        pltpu.sync_copy(acc.at[pl.ds(s * WR, WR)],
                        out_hbm.at[pl.ds(chunk * NPAD + nbase + s * WR, WR)])
        plsc.subcore_barrier()


_agg_call = pl.kernel(
    _agg_body,
    out_type=jax.ShapeDtypeStruct((NC * NPAD, CW), f32),
    mesh=plsc.VectorSubcoreMesh(core_axis_name="c", subcore_axis_name="s"),
    scratch_types=[
        pltpu.VMEM((EB,), jnp.int32),
        pltpu.VMEM((EB,), jnp.int32),
        pltpu.VMEM((EB, CW), f32),
        pltpu.VMEM((16,), jnp.int32),
        pltpu.VMEM_SHARED((ACCR, CW), f32),
        pltpu.SemaphoreType.DMA,
    ],
    name="sc_segment_sum",
)


def _proj_body(x_ref, w_ref, p_ref):
    p = jnp.dot(x_ref[...], w_ref[...], preferred_element_type=f32)
    for c in range(NC):
        p_ref[c] = p[:, c * CW:(c + 1) * CW]


def _stats_accum(i, y, acc_s, acc_ss, st_ref):
    @pl.when(i == 0)
    def _():
        acc_s[...] = jnp.zeros_like(acc_s)
        acc_ss[...] = jnp.zeros_like(acc_ss)

    acc_s[...] += jnp.sum(y, axis=0, keepdims=True)
    acc_ss[...] += jnp.sum(y * y, axis=0, keepdims=True)

    @pl.when(i == NBLK - 1)
    def _():
        st_ref[0:1, :] = acc_s[...]
        st_ref[1:2, :] = acc_ss[...]


def _addagg_body(p_ref, agg_ref, b1_ref, eps_ref, y_ref, st_ref, acc_s, acc_ss):
    i = pl.program_id(0)
    ep = 1.0 + eps_ref[0, 0]
    parts = [ep * p_ref[c] + agg_ref[c] for c in range(NC)]
    y = jnp.concatenate(parts, axis=1) + b1_ref[...]
    y_ref[...] = y
    _stats_accum(i, y, acc_s, acc_ss, st_ref)


def _conv_mm1_body(h_ref, agg_ref, w1_ref, b1_ref, eps_ref,
                   y_ref, st_ref, acc_s, acc_ss):
    i = pl.program_id(0)
    ep = 1.0 + eps_ref[0, 0]
    y = jnp.zeros((BR, H), f32)
    for c in range(NC):
        z = ep * h_ref[c] + agg_ref[c]
        y += jnp.dot(z, w1_ref[pl.ds(c * CW, CW), :], preferred_element_type=f32)
    y = y + b1_ref[...]
    y_ref[...] = y
    _stats_accum(i, y, acc_s, acc_ss, st_ref)


def _norm_coefs(st_ref, g_ref, be_ref):
    m = st_ref[0:1, :] * (1.0 / N)
    ey2 = st_ref[1:2, :] * (1.0 / N)
    var = ey2 - m * m
    rstd = lax.rsqrt(var + 1e-5)
    scale = rstd * g_ref[...]
    shift = be_ref[...] - m * scale
    return scale, shift


def _conv_mm2_body(y1_ref, st_ref, g_ref, be_ref, w2_ref, b2_ref,
                   y2_ref, st2_ref, acc_s, acc_ss):
    i = pl.program_id(0)
    scale, shift = _norm_coefs(st_ref, g_ref, be_ref)
    z = jnp.maximum(y1_ref[...] * scale + shift, 0.0)
    y = jnp.dot(z, w2_ref[...], preferred_element_type=f32) + b2_ref[...]
    y2_ref[...] = y
    _stats_accum(i, y, acc_s, acc_ss, st2_ref)


def _norm_relu_body(y2_ref, st_ref, g_ref, be_ref, h_ref):
    scale, shift = _norm_coefs(st_ref, g_ref, be_ref)
    z = jnp.maximum(y2_ref[...] * scale + shift, 0.0)
    for c in range(NC):
        h_ref[c] = z[:, c * CW:(c + 1) * CW]


def _final_body(h1_ref, h2_ref, h3_ref, h4_ref, batch_ref,
                w_ref, b_ref, pw_ref, pb_ref, out_ref, pool_ref):
    i = pl.program_id(0)
    hrefs = [h1_ref, h2_ref, h3_ref, h4_ref]
    hl = jnp.zeros((BR, H), f32)
    for l in range(L):
        for c in range(NC):
            k = l * NC + c
            hl += jnp.dot(hrefs[l][c], w_ref[pl.ds(k * CW, CW), :],
                          preferred_element_type=f32)
    hl = jnp.maximum(hl + b_ref[...], 0.0)
    bvals = batch_ref[0, 0, :].reshape(BR, 1)
    gids = lax.broadcasted_iota(jnp.int32, (1, G), 1)
    onehot = (bvals == gids).astype(f32)
    contrib = lax.dot_general(onehot, hl, (((0,), (0,)), ((), ())),
                              preferred_element_type=f32)

    @pl.when(i == 0)
    def _():
        pool_ref[...] = jnp.zeros_like(pool_ref)

    pool_ref[...] += contrib

    @pl.when(i == NBLK - 1)
    def _():
        out_ref[...] = (jnp.dot(pool_ref[...], pw_ref[...],
                                preferred_element_type=f32) + pb_ref[...])


def _chunk_spec():
    return pl.BlockSpec((NC, BR, CW), lambda i: (0, i, 0))


def _agg_spec():
    return pl.BlockSpec((NC, BR, CW), lambda i: (0, i, 0))


def _row_spec():
    return pl.BlockSpec((BR, H), lambda i: (i, 0))


def _full(shape):
    return pl.BlockSpec(shape, lambda i: tuple(0 for _ in shape))


def _stats_scratch():
    return [pltpu.VMEM((1, H), f32), pltpu.VMEM((1, H), f32)]


_proj = pl.pallas_call(
    _proj_body,
    grid=(NBLK,),
    in_specs=[pl.BlockSpec((BR, F_IN), lambda i: (i, 0)), _full((F_IN, H))],
    out_specs=_chunk_spec(),
    out_shape=jax.ShapeDtypeStruct((NC, N, CW), f32),
)

_addagg = pl.pallas_call(
    _addagg_body,
    grid=(NBLK,),
    in_specs=[_chunk_spec(), _agg_spec(), _full((1, H)), _full((1, 1))],
    out_specs=[_row_spec(), _full((2, H))],
    out_shape=[jax.ShapeDtypeStruct((N, H), f32),
               jax.ShapeDtypeStruct((2, H), f32)],
    scratch_shapes=_stats_scratch(),
)

_conv_mm1 = pl.pallas_call(
    _conv_mm1_body,
    grid=(NBLK,),
    in_specs=[_chunk_spec(), _agg_spec(), _full((H, H)), _full((1, H)),
              _full((1, 1))],
    out_specs=[_row_spec(), _full((2, H))],
    out_shape=[jax.ShapeDtypeStruct((N, H), f32),
               jax.ShapeDtypeStruct((2, H), f32)],
    scratch_shapes=_stats_scratch(),
)

_conv_mm2 = pl.pallas_call(
    _conv_mm2_body,
    grid=(NBLK,),
    in_specs=[_row_spec(), _full((2, H)), _full((1, H)), _full((1, H)),
              _full((H, H)), _full((1, H))],
    out_specs=[_row_spec(), _full((2, H))],
    out_shape=[jax.ShapeDtypeStruct((N, H), f32),
               jax.ShapeDtypeStruct((2, H), f32)],
    scratch_shapes=_stats_scratch(),
)

_norm_relu = pl.pallas_call(
    _norm_relu_body,
    grid=(NBLK,),
    in_specs=[_row_spec(), _full((2, H)), _full((1, H)), _full((1, H))],
    out_specs=_chunk_spec(),
    out_shape=jax.ShapeDtypeStruct((NC, N, CW), f32),
)

_final = pl.pallas_call(
    _final_body,
    grid=(NBLK,),
    in_specs=[_chunk_spec(), _chunk_spec(), _chunk_spec(), _chunk_spec(),
              pl.BlockSpec((1, 1, BR), lambda i: (i, 0, 0)),
              _full((L * H, H)), _full((1, H)), _full((H, C)), _full((1, C))],
    out_specs=_full((G, C)),
    out_shape=jax.ShapeDtypeStruct((G, C), f32),
    scratch_shapes=[pltpu.VMEM((G, H), f32)],
)


def _row(v):
    return v.reshape(1, -1)


def kernel(x, edge_index, batch, params):
    src = edge_index[0]
    dst = edge_index[1]
    psrc, pdst, nb = _part_call(src, dst)
    zeros = jnp.zeros((ZR, CW), f32)
    batch_r = batch.reshape(NBLK, 1, BR)

    def run_agg(h_chunks):
        flat = h_chunks.reshape(NC * N, CW)
        return _agg_call(flat, psrc, pdst, nb, zeros).reshape(NC, NPAD, CW)

    hs = []
    h_chunks = None
    for li in range(L):
        p = params['convs'][li]
        eps = p['eps'].reshape(1, 1)
        if li == 0:
            pch = _proj(x, p['W1'])
            agg = run_agg(pch)
            y1, st1 = _addagg(pch, agg, _row(p['b1']), eps)
        else:
            agg = run_agg(h_chunks)
            y1, st1 = _conv_mm1(h_chunks, agg, p['W1'], _row(p['b1']), eps)
        y2, st2 = _conv_mm2(y1, st1, _row(p['g1']), _row(p['be1']),
                            p['W2'], _row(p['b2']))
        h_chunks = _norm_relu(y2, st2, _row(p['g2']), _row(p['be2']))
        hs.append(h_chunks)

    out = _final(hs[0], hs[1], hs[2], hs[3], batch_r,
                 params['lin1_W'], _row(params['lin1_b']),
                 params['pred_W'], _row(params['pred_b']))
    return out


# speedup vs baseline: 3.5114x; 1.0878x over previous
import functools

import jax
import jax.numpy as jnp
from jax import lax
from jax.experimental import pallas as pl
from jax.experimental.pallas import tpu as pltpu
from jax.experimental.pallas import tpu_sc as plsc

N = 10000
E = 160000
F_IN = 5
H = 512
L = 4
C = 10
G = 64

NC = 4
CW = H // NC
BR = 2000
NBLK = N // BR

NSC = 2
NSUB = 16
EPW = E // NSUB
EB = 400
NEB = EPW // EB
NPAD = 10240
HALF = NPAD // 2
JROWS = NSUB * 16
ACCR = HALF + JROWS
ZR = ACCR // NSUB
WR = HALF // NSUB
EC = E + NSUB * EB
BMAX = EC // EB
PCAP = EPW + EB

f32 = jnp.float32


def _part_body(src_hbm, dst_hbm, psrc_hbm, pdst_hbm, nb_hbm,
               src_v, dst_v, bs, bd, stage_v, tbl_v, table):
    c = lax.axis_index("c")
    s = lax.axis_index("s")
    lane = lax.iota(jnp.int32, 16)
    keep_low = c == 0
    ebase = s * EPW

    @pl.loop(0, PCAP // 16)
    def _fill(i):
        bs[pl.ds(i * 16, 16)] = lane
        bd[pl.ds(i * 16, 16)] = jnp.full((16,), 1 << 20, jnp.int32)

    @pl.loop(0, NEB, init_carry=jnp.zeros((16,), jnp.int32))
    def _count(b, cnt):
        pltpu.sync_copy(dst_hbm.at[pl.ds(ebase + b * EB, EB)], dst_v)
        for j in range(EB // 16):
            low = dst_v[pl.ds(j * 16, 16)] < HALF
            m = jnp.where(keep_low, low, ~low)
            cnt = cnt + plsc.all_reduce_population_count(m)
        return cnt

    stage_v[...] = _count
    pltpu.sync_copy(stage_v, table.at[pl.ds(s * 16, 16)])
    plsc.subcore_barrier()
    pltpu.sync_copy(table, tbl_v)

    nblk_before = jnp.int32(0)
    nblk_total = jnp.int32(0)
    for j in range(NSUB):
        bj = (tbl_v[pl.ds(j * 16, 16)][0] + (EB - 1)) // EB
        nblk_total = nblk_total + bj
        nblk_before = nblk_before + jnp.where(s > j, bj, 0)
    my_off = c * EC + nblk_before * EB

    @pl.loop(0, NEB, init_carry=jnp.int32(0))
    def _compact(b, off):
        pltpu.sync_copy(src_hbm.at[pl.ds(ebase + b * EB, EB)], src_v)
        pltpu.sync_copy(dst_hbm.at[pl.ds(ebase + b * EB, EB)], dst_v)
        for j in range(EB // 16):
            d16 = dst_v[pl.ds(j * 16, 16)]
            s16 = src_v[pl.ds(j * 16, 16)]
            low = d16 < HALF
            m = jnp.where(keep_low, low, ~low)
            plsc.store_compressed(bs.at[pl.ds(off, 16)], s16, mask=m)
            plsc.store_compressed(bd.at[pl.ds(off, 16)], d16, mask=m)
            stage_v[...] = plsc.all_reduce_population_count(m)
            off = off + stage_v[...][0]
        return off

    bs[pl.ds(_compact, 16)] = lane
    bd[pl.ds(_compact, 16)] = jnp.full((16,), 1 << 20, jnp.int32)

    nblk = (_compact + (EB - 1)) // EB

    @pl.loop(0, nblk)
    def _flush(b):
        pltpu.sync_copy(bs.at[pl.ds(b * EB, EB)],
                        psrc_hbm.at[pl.ds(my_off + b * EB, EB)])
        pltpu.sync_copy(bd.at[pl.ds(b * EB, EB)],
                        pdst_hbm.at[pl.ds(my_off + b * EB, EB)])

    @pl.when(s == 0)
    def _():
        stage_v[...] = jnp.where(lane == 0, nblk_total, 0)
        pltpu.sync_copy(stage_v, nb_hbm.at[pl.ds(c * 16, 16)])


_part_call = pl.kernel(
    _part_body,
    out_type=(jax.ShapeDtypeStruct((NSC * EC,), jnp.int32),
              jax.ShapeDtypeStruct((NSC * EC,), jnp.int32),
              jax.ShapeDtypeStruct((NSC * 16,), jnp.int32)),
    mesh=plsc.VectorSubcoreMesh(core_axis_name="c", subcore_axis_name="s"),
    compiler_params=pltpu.CompilerParams(needs_layout_passes=False),
    scratch_types=[
        pltpu.VMEM((EB,), jnp.int32),
        pltpu.VMEM((EB,), jnp.int32),
        pltpu.VMEM((PCAP,), jnp.int32),
        pltpu.VMEM((PCAP,), jnp.int32),
        pltpu.VMEM((16,), jnp.int32),
        pltpu.VMEM((NSUB * 16,), jnp.int32),
        pltpu.VMEM_SHARED((NSUB * 16,), jnp.int32),
    ],
    name="sc_edge_partition",
)


def _agg_body(h_hbm, psrc_hbm, pdst_hbm, nb_hbm, zeros_hbm, out_hbm,
              esrc, edst, rows_v0, nb_v, acc, sem0):
    c = lax.axis_index("c")
    s = lax.axis_index("s")
    lane = lax.iota(jnp.int32, 16)
    nbase = c * HALF
    junk = HALF + s * 16 + lane
    pltpu.sync_copy(nb_hbm.at[pl.ds(c * 16, 16)], nb_v)
    bg = nb_v[...][0]
    bt = (bg + (NSUB - 1)) // NSUB
    lo = jnp.minimum(s * bt, bg)
    myb = jnp.minimum(lo + bt, bg) - lo
    gbase = c * EC

    @pl.loop(0, BTMAX)
    def _preload(i):
        @pl.when(i < myb)
        def _():
            off = gbase + (lo + i) * EB
            pltpu.sync_copy(psrc_hbm.at[pl.ds(off, EB)], esrc.at[pl.ds(i * EB, EB)])
            pltpu.sync_copy(pdst_hbm.at[pl.ds(off, EB)], edst.at[pl.ds(i * EB, EB)])
            for j in range(EB // 16):
                d = edst[pl.ds(i * EB + j * 16, 16)] - nbase
                ok = (d >= 0) & (d < HALF)
                edst[pl.ds(i * EB + j * 16, 16)] = jnp.where(ok, d, junk)

    for chunk in range(NC):
        if chunk > 0:
            @pl.loop(0, BTMAX)
            def _shift(i):
                @pl.when(i < myb)
                def _():
                    for j in range(EB // 16):
                        esrc[pl.ds(i * EB + j * 16, 16)] = (
                            esrc[pl.ds(i * EB + j * 16, 16)] + N)

        pltpu.sync_copy(zeros_hbm, acc.at[pl.ds(s * ZR, ZR)])
        plsc.subcore_barrier()

        @pl.loop(0, BTMAX)
        def _edge_block(b):
            @pl.when(b < myb)
            def _():
                pltpu.async_copy(h_hbm.at[esrc.at[pl.ds(b * EB, EB)]],
                                 rows_v0, sem0).wait()
                pltpu.sync_copy(rows_v0, acc.at[edst.at[pl.ds(b * EB, EB)]],
                                add=True)

        plsc.subcore_barrier()
        pltpu.sync_copy(acc.at[pl.ds(s * WR, WR)],
                        out_hbm.at[pl.ds(chunk * NPAD + nbase + s * WR, WR)])
        plsc.subcore_barrier()


BTMAX = (BMAX + NSUB - 1) // NSUB

_agg_call = pl.kernel(
    _agg_body,
    out_type=jax.ShapeDtypeStruct((NC * NPAD, CW), f32),
    mesh=plsc.VectorSubcoreMesh(core_axis_name="c", subcore_axis_name="s"),
    scratch_types=[
        pltpu.VMEM((BTMAX * EB,), jnp.int32),
        pltpu.VMEM((BTMAX * EB,), jnp.int32),
        pltpu.VMEM((EB, CW), f32),
        pltpu.VMEM((16,), jnp.int32),
        pltpu.VMEM_SHARED((ACCR, CW), f32),
        pltpu.SemaphoreType.DMA,
    ],
    name="sc_segment_sum",
)


def _proj_body(x_ref, w_ref, p_ref):
    p = jnp.dot(x_ref[...], w_ref[...], preferred_element_type=f32)
    for c in range(NC):
        p_ref[c] = p[:, c * CW:(c + 1) * CW]


def _stats_accum(i, y, acc_s, acc_ss, st_ref):
    @pl.when(i == 0)
    def _():
        acc_s[...] = jnp.zeros_like(acc_s)
        acc_ss[...] = jnp.zeros_like(acc_ss)

    acc_s[...] += jnp.sum(y, axis=0, keepdims=True)
    acc_ss[...] += jnp.sum(y * y, axis=0, keepdims=True)

    @pl.when(i == NBLK - 1)
    def _():
        st_ref[0:1, :] = acc_s[...]
        st_ref[1:2, :] = acc_ss[...]


def _addagg_body(p_ref, agg_ref, b1_ref, eps_ref, y_ref, st_ref, acc_s, acc_ss):
    i = pl.program_id(0)
    ep = 1.0 + eps_ref[0, 0]
    parts = [ep * p_ref[c] + agg_ref[c] for c in range(NC)]
    y = jnp.concatenate(parts, axis=1) + b1_ref[...]
    y_ref[...] = y
    _stats_accum(i, y, acc_s, acc_ss, st_ref)


def _conv_mm1_body(h_ref, agg_ref, w1_ref, b1_ref, eps_ref,
                   y_ref, st_ref, acc_s, acc_ss):
    i = pl.program_id(0)
    ep = 1.0 + eps_ref[0, 0]
    y = jnp.zeros((BR, H), f32)
    for c in range(NC):
        z = ep * h_ref[c] + agg_ref[c]
        y += jnp.dot(z, w1_ref[pl.ds(c * CW, CW), :], preferred_element_type=f32)
    y = y + b1_ref[...]
    y_ref[...] = y
    _stats_accum(i, y, acc_s, acc_ss, st_ref)


def _norm_coefs(st_ref, g_ref, be_ref):
    m = st_ref[0:1, :] * (1.0 / N)
    ey2 = st_ref[1:2, :] * (1.0 / N)
    var = ey2 - m * m
    rstd = lax.rsqrt(var + 1e-5)
    scale = rstd * g_ref[...]
    shift = be_ref[...] - m * scale
    return scale, shift


def _conv_mm2_body(y1_ref, st_ref, g_ref, be_ref, w2_ref, b2_ref,
                   y2_ref, st2_ref, acc_s, acc_ss):
    i = pl.program_id(0)
    scale, shift = _norm_coefs(st_ref, g_ref, be_ref)
    z = jnp.maximum(y1_ref[...] * scale + shift, 0.0)
    y = jnp.dot(z, w2_ref[...], preferred_element_type=f32) + b2_ref[...]
    y2_ref[...] = y
    _stats_accum(i, y, acc_s, acc_ss, st2_ref)


def _norm_relu_body(y2_ref, st_ref, g_ref, be_ref, h_ref):
    scale, shift = _norm_coefs(st_ref, g_ref, be_ref)
    z = jnp.maximum(y2_ref[...] * scale + shift, 0.0)
    for c in range(NC):
        h_ref[c] = z[:, c * CW:(c + 1) * CW]


def _final_body(h1_ref, h2_ref, h3_ref, h4_ref, batch_ref,
                w_ref, b_ref, pw_ref, pb_ref, out_ref, pool_ref):
    i = pl.program_id(0)
    hrefs = [h1_ref, h2_ref, h3_ref, h4_ref]
    hl = jnp.zeros((BR, H), f32)
    for l in range(L):
        for c in range(NC):
            k = l * NC + c
            hl += jnp.dot(hrefs[l][c], w_ref[pl.ds(k * CW, CW), :],
                          preferred_element_type=f32)
    hl = jnp.maximum(hl + b_ref[...], 0.0)
    bvals = batch_ref[0, 0, :].reshape(BR, 1)
    gids = lax.broadcasted_iota(jnp.int32, (1, G), 1)
    onehot = (bvals == gids).astype(f32)
    contrib = lax.dot_general(onehot, hl, (((0,), (0,)), ((), ())),
                              preferred_element_type=f32)

    @pl.when(i == 0)
    def _():
        pool_ref[...] = jnp.zeros_like(pool_ref)

    pool_ref[...] += contrib

    @pl.when(i == NBLK - 1)
    def _():
        out_ref[...] = (jnp.dot(pool_ref[...], pw_ref[...],
                                preferred_element_type=f32) + pb_ref[...])


def _chunk_spec():
    return pl.BlockSpec((NC, BR, CW), lambda i: (0, i, 0))


def _agg_spec():
    return pl.BlockSpec((NC, BR, CW), lambda i: (0, i, 0))


def _row_spec():
    return pl.BlockSpec((BR, H), lambda i: (i, 0))


def _full(shape):
    return pl.BlockSpec(shape, lambda i: tuple(0 for _ in shape))


def _stats_scratch():
    return [pltpu.VMEM((1, H), f32), pltpu.VMEM((1, H), f32)]


_proj = pl.pallas_call(
    _proj_body,
    grid=(NBLK,),
    in_specs=[pl.BlockSpec((BR, F_IN), lambda i: (i, 0)), _full((F_IN, H))],
    out_specs=_chunk_spec(),
    out_shape=jax.ShapeDtypeStruct((NC, N, CW), f32),
)

_addagg = pl.pallas_call(
    _addagg_body,
    grid=(NBLK,),
    in_specs=[_chunk_spec(), _agg_spec(), _full((1, H)), _full((1, 1))],
    out_specs=[_row_spec(), _full((2, H))],
    out_shape=[jax.ShapeDtypeStruct((N, H), f32),
               jax.ShapeDtypeStruct((2, H), f32)],
    scratch_shapes=_stats_scratch(),
)

_conv_mm1 = pl.pallas_call(
    _conv_mm1_body,
    grid=(NBLK,),
    in_specs=[_chunk_spec(), _agg_spec(), _full((H, H)), _full((1, H)),
              _full((1, 1))],
    out_specs=[_row_spec(), _full((2, H))],
    out_shape=[jax.ShapeDtypeStruct((N, H), f32),
               jax.ShapeDtypeStruct((2, H), f32)],
    scratch_shapes=_stats_scratch(),
)

_conv_mm2 = pl.pallas_call(
    _conv_mm2_body,
    grid=(NBLK,),
    in_specs=[_row_spec(), _full((2, H)), _full((1, H)), _full((1, H)),
              _full((H, H)), _full((1, H))],
    out_specs=[_row_spec(), _full((2, H))],
    out_shape=[jax.ShapeDtypeStruct((N, H), f32),
               jax.ShapeDtypeStruct((2, H), f32)],
    scratch_shapes=_stats_scratch(),
)

_norm_relu = pl.pallas_call(
    _norm_relu_body,
    grid=(NBLK,),
    in_specs=[_row_spec(), _full((2, H)), _full((1, H)), _full((1, H))],
    out_specs=_chunk_spec(),
    out_shape=jax.ShapeDtypeStruct((NC, N, CW), f32),
)

_final = pl.pallas_call(
    _final_body,
    grid=(NBLK,),
    in_specs=[_chunk_spec(), _chunk_spec(), _chunk_spec(), _chunk_spec(),
              pl.BlockSpec((1, 1, BR), lambda i: (i, 0, 0)),
              _full((L * H, H)), _full((1, H)), _full((H, C)), _full((1, C))],
    out_specs=_full((G, C)),
    out_shape=jax.ShapeDtypeStruct((G, C), f32),
    scratch_shapes=[pltpu.VMEM((G, H), f32)],
)


def _row(v):
    return v.reshape(1, -1)


def kernel(x, edge_index, batch, params):
    src = edge_index[0]
    dst = edge_index[1]
    psrc, pdst, nb = _part_call(src, dst)
    zeros = jnp.zeros((ZR, CW), f32)
    batch_r = batch.reshape(NBLK, 1, BR)

    def run_agg(h_chunks):
        flat = h_chunks.reshape(NC * N, CW)
        return _agg_call(flat, psrc, pdst, nb, zeros).reshape(NC, NPAD, CW)

    hs = []
    h_chunks = None
    for li in range(L):
        p = params['convs'][li]
        eps = p['eps'].reshape(1, 1)
        if li == 0:
            pch = _proj(x, p['W1'])
            agg = run_agg(pch)
            y1, st1 = _addagg(pch, agg, _row(p['b1']), eps)
        else:
            agg = run_agg(h_chunks)
            y1, st1 = _conv_mm1(h_chunks, agg, p['W1'], _row(p['b1']), eps)
        y2, st2 = _conv_mm2(y1, st1, _row(p['g1']), _row(p['be1']),
                            p['W2'], _row(p['b2']))
        h_chunks = _norm_relu(y2, st2, _row(p['g2']), _row(p['be2']))
        hs.append(h_chunks)

    out = _final(hs[0], hs[1], hs[2], hs[3], batch_r,
                 params['lin1_W'], _row(params['lin1_b']),
                 params['pred_W'], _row(params['pred_b']))
    return out
